# baseline (speedup 1.0000x reference)
import jax
import jax.numpy as jnp
from jax import lax
from jax.experimental import pallas as pl
from jax.experimental.pallas import tpu as pltpu
from jax.experimental.pallas import tpu_sc as plsc

L = 16
NC = 2
NS = 16
NW = NC * NS
B = 8
H = 224
W = 224
HW = H * W
TR = H // 8
UPW = B * TR // NW
NROWVEC = W // L
NQ = 5


_LOG1P_COEF = (
    -0.006006605050865348, 0.03426459995555095, -0.09229041738055285,
    0.16499812983410006, -0.23943337074600235, 0.33144665224343317,
    -0.49982549864347925, 0.9999936302585147, 3.910905554960209e-08,
)


def _bce(x, t):
    e = jnp.exp(-jnp.abs(x))
    p = jnp.float32(_LOG1P_COEF[0])
    for c in _LOG1P_COEF[1:]:
        p = p * e + jnp.float32(c)
    return jnp.maximum(x, 0.0) - x * t + p


def _sc_body(pred_hbm, target_hbm, out_hbm, buf, stage, sem0, sem1):
    wid = lax.axis_index("s") * NC + lax.axis_index("c")
    u0 = wid * UPW
    smp = u0 // TR
    row_base = (u0 - smp * TR) * 8
    sems = (sem0, sem1)
    zero = jnp.zeros((L,), jnp.float32)

    def issue(k):
        slot = k % 2
        r0 = row_base + k * 8
        handles = []
        for c in range(3):
            handles.append(pltpu.async_copy(
                pred_hbm.at[smp, c, pl.ds(r0, 8), :], buf.at[slot, c],
                sems[slot]))
            handles.append(pltpu.async_copy(
                target_hbm.at[smp, c, pl.ds(r0, 8), :], buf.at[slot, 3 + c],
                sems[slot]))
        return handles

    inflight = {0: issue(0), 1: issue(1)}
    accs = (zero, zero, zero, zero, zero)

    for k in range(UPW):
        slot = k % 2
        for h in inflight.pop(k):
            h.wait()
        if k + 2 < UPW:
            inflight[k + 2] = issue(k + 2)

        def row_step(r, carry, slot=slot):
            def step(j, carry, slot=slot, r=r):
                acc_t2, acc_c, acc_ti, acc_w, acc_wc = carry
                o = j * L
                p0 = buf[slot, 0, r, pl.ds(o, L)]
                p1 = buf[slot, 1, r, pl.ds(o, L)]
                p2 = buf[slot, 2, r, pl.ds(o, L)]
                t0 = buf[slot, 3, r, pl.ds(o, L)]
                t1 = buf[slot, 4, r, pl.ds(o, L)]
                t2 = buf[slot, 5, r, pl.ds(o, L)]
                tc = t2 - t2 * t0
                roi = 1.0 - tc * (1.0 - t1)
                wm = jnp.where(roi != 0.0, 1.0, 0.0)
                acc_t2 = acc_t2 + t2
                acc_c = acc_c + _bce(p2, tc)
                acc_ti = acc_ti + _bce(p0, t0)
                acc_w = acc_w + _bce(p1, t1) * wm
                acc_wc = acc_wc + wm
                return acc_t2, acc_c, acc_ti, acc_w, acc_wc

            return lax.fori_loop(0, NROWVEC, step, carry)

        accs = lax.fori_loop(0, 8, row_step, accs)

    for q in range(NQ):
        stage[q] = accs[q]
    pltpu.sync_copy(stage, out_hbm.at[wid])


def _combine_body(part_ref, o_ref):
    x = part_ref[...]
    t = jnp.sum(jnp.sum(x, axis=1), axis=-1)
    li = lax.broadcasted_iota(jnp.int32, (B, NQ), 1)
    sel = t[:, :1] != 0.0
    zero = jnp.zeros_like(t)
    cnum = jnp.sum(jnp.where((li == 1) & sel, t, zero))
    cden = jnp.sum(jnp.where((li == 1) & sel, jnp.float32(HW), zero))
    tsum = jnp.sum(jnp.where(li == 2, t, zero))
    wsum = jnp.sum(jnp.where(li == 3, t, zero))
    wden = jnp.sum(jnp.where(li == 4, t, zero))
    centroid = jnp.where(cden > 0.0, cnum / jnp.maximum(cden, 1.0), 0.0)
    tissue = tsum / jnp.float32(B * HW)
    whole = jnp.where(wden > 0.0, wsum / jnp.maximum(wden, 1.0), 0.0)
    o_ref[...] = jnp.reshape(whole + centroid + tissue, (1, 1))


@jax.jit
def kernel(pred, target):
    mesh = plsc.VectorSubcoreMesh(core_axis_name="c", subcore_axis_name="s",
                                  num_cores=NC, num_subcores=NS)
    partials = pl.kernel(
        _sc_body,
        out_type=jax.ShapeDtypeStruct((NW, NQ, L), jnp.float32),
        mesh=mesh,
        scratch_types=[
            pltpu.VMEM((2, 6, 8, W), jnp.float32),
            pltpu.VMEM((NQ, L), jnp.float32),
            pltpu.SemaphoreType.DMA,
            pltpu.SemaphoreType.DMA,
        ],
        compiler_params=pltpu.CompilerParams(use_tc_tiling_on_sc=True),
    )(pred, target)
    out = pl.pallas_call(
        _combine_body,
        out_shape=jax.ShapeDtypeStruct((1, 1), jnp.float32),
    )(partials.reshape(B, NW // B, NQ, L))
    return out[0, 0]


# speedup vs baseline: 1.1032x; 1.1032x over previous
import jax
import jax.numpy as jnp
from jax import lax
from jax.experimental import pallas as pl
from jax.experimental.pallas import tpu as pltpu
from jax.experimental.pallas import tpu_sc as plsc

L = 16
NC = 2
NS = 16
NW = NC * NS
B = 8
H = 224
W = 224
HW = H * W
TR = H // 8
UPW = B * TR // NW
NROWVEC = W // L
NQ = 4

_LOG1P_COEF = (
    -0.006006605050865348, 0.03426459995555095, -0.09229041738055285,
    0.16499812983410006, -0.23943337074600235, 0.33144665224343317,
    -0.49982549864347925, 0.9999936302585147, 3.910905554960209e-08,
)


def _bce(x, t):
    e = jnp.exp(-jnp.abs(x))
    p = jnp.float32(_LOG1P_COEF[0])
    for c in _LOG1P_COEF[1:]:
        p = p * e + jnp.float32(c)
    return jnp.maximum(x, 0.0) - x * t + p


def _sc_body(pred_hbm, target_hbm, out_hbm, buf, stage, sem0, sem1):
    wid = lax.axis_index("s") * NC + lax.axis_index("c")
    u0 = wid * UPW
    smp = u0 // TR
    row_base = (u0 - smp * TR) * 8
    sems = (sem0, sem1)
    zero = jnp.zeros((L,), jnp.float32)

    def issue(k):
        slot = k % 2
        r0 = row_base + k * 8
        handles = []
        for c in (1, 2):
            handles.append(pltpu.async_copy(
                pred_hbm.at[smp, c, pl.ds(r0, 8), :], buf.at[slot, c - 1],
                sems[slot]))
        for c in (0, 1, 2):
            handles.append(pltpu.async_copy(
                target_hbm.at[smp, c, pl.ds(r0, 8), :], buf.at[slot, 2 + c],
                sems[slot]))
        return handles

    inflight = {0: issue(0), 1: issue(1)}
    accs = (zero, zero, zero, zero)

    for k in range(UPW):
        slot = k % 2
        for h in inflight.pop(k):
            h.wait()
        if k + 2 < UPW:
            inflight[k + 2] = issue(k + 2)

        def row_step(r, carry, slot=slot):
            def step(j, carry, slot=slot, r=r):
                acc_t2, acc_c, acc_w, acc_wc = carry
                o = j * L
                p1 = buf[slot, 0, r, pl.ds(o, L)]
                p2 = buf[slot, 1, r, pl.ds(o, L)]
                t0 = buf[slot, 2, r, pl.ds(o, L)]
                t1 = buf[slot, 3, r, pl.ds(o, L)]
                t2 = buf[slot, 4, r, pl.ds(o, L)]
                tc = t2 - t2 * t0
                roi = 1.0 - tc * (1.0 - t1)
                wm = jnp.where(roi != 0.0, 1.0, 0.0)
                acc_t2 = acc_t2 + t2
                acc_c = acc_c + _bce(p2, tc)
                acc_w = acc_w + _bce(p1, t1) * wm
                acc_wc = acc_wc + wm
                return acc_t2, acc_c, acc_w, acc_wc

            return lax.fori_loop(0, NROWVEC, step, carry)

        accs = lax.fori_loop(0, 8, row_step, accs)

    for q in range(NQ):
        stage[q] = accs[q]
    pltpu.sync_copy(stage, out_hbm.at[wid])


def _tissue_body(p_ref, t_ref, o_ref):
    x = p_ref[...]
    t = t_ref[...]
    b = jnp.maximum(x, 0.0) - x * t + jnp.log1p(jnp.exp(-jnp.abs(x)))
    o_ref[0, 0] = jnp.sum(b)


def _combine_body(part_ref, tis_ref, o_ref):
    x = part_ref[...]
    t = jnp.sum(jnp.sum(x, axis=1), axis=-1)
    li = lax.broadcasted_iota(jnp.int32, (B, NQ), 1)
    sel = t[:, :1] != 0.0
    zero = jnp.zeros_like(t)
    cnum = jnp.sum(jnp.where((li == 1) & sel, t, zero))
    cden = jnp.sum(jnp.where((li == 1) & sel, jnp.float32(HW), zero))
    wsum = jnp.sum(jnp.where(li == 2, t, zero))
    wden = jnp.sum(jnp.where(li == 3, t, zero))
    centroid = jnp.where(cden > 0.0, cnum / jnp.maximum(cden, 1.0), 0.0)
    tissue = tis_ref[0, 0] / jnp.float32(B * HW)
    whole = jnp.where(wden > 0.0, wsum / jnp.maximum(wden, 1.0), 0.0)
    o_ref[0, 0] = whole + centroid + tissue


@jax.jit
def kernel(pred, target):
    mesh = plsc.VectorSubcoreMesh(core_axis_name="c", subcore_axis_name="s",
                                  num_cores=NC, num_subcores=NS)
    partials = pl.kernel(
        _sc_body,
        out_type=jax.ShapeDtypeStruct((NW, NQ, L), jnp.float32),
        mesh=mesh,
        scratch_types=[
            pltpu.VMEM((2, 5, 8, W), jnp.float32),
            pltpu.VMEM((NQ, L), jnp.float32),
            pltpu.SemaphoreType.DMA,
            pltpu.SemaphoreType.DMA,
        ],
        compiler_params=pltpu.CompilerParams(use_tc_tiling_on_sc=True),
    )(pred, target)
    tissue = pl.pallas_call(
        _tissue_body,
        grid=(1,),
        in_specs=[
            pl.BlockSpec((B, 1, H, W), lambda i: (0, 0, 0, 0)),
            pl.BlockSpec((B, 1, H, W), lambda i: (0, 0, 0, 0)),
        ],
        out_specs=pl.BlockSpec(memory_space=pltpu.SMEM),
        out_shape=jax.ShapeDtypeStruct((1, 1), jnp.float32),
    )(pred, target)
    out = pl.pallas_call(
        _combine_body,
        in_specs=[
            pl.BlockSpec((B, NW // B, NQ, L), lambda: (0, 0, 0, 0)),
            pl.BlockSpec(memory_space=pltpu.SMEM),
        ],
        out_specs=pl.BlockSpec(memory_space=pltpu.SMEM),
        out_shape=jax.ShapeDtypeStruct((1, 1), jnp.float32),
    )(partials.reshape(B, NW // B, NQ, L), tissue)
    return out[0, 0]


# speedup vs baseline: 1.1136x; 1.0095x over previous
import jax
import jax.numpy as jnp
from jax import lax
from jax.experimental import pallas as pl
from jax.experimental.pallas import tpu as pltpu
from jax.experimental.pallas import tpu_sc as plsc

L = 16
NC = 2
NS = 16
NW = NC * NS
B = 8
H = 224
W = 224
HW = H * W
TR = H // 8
UPW = B * TR // NW
NROWVEC = W // L
NQ = 4

_LOG1P_COEF = (
    -0.006006605050865348, 0.03426459995555095, -0.09229041738055285,
    0.16499812983410006, -0.23943337074600235, 0.33144665224343317,
    -0.49982549864347925, 0.9999936302585147, 3.910905554960209e-08,
)


def _bce(x, t):
    e = jnp.exp(-jnp.abs(x))
    p = jnp.float32(_LOG1P_COEF[0])
    for c in _LOG1P_COEF[1:]:
        p = p * e + jnp.float32(c)
    return jnp.maximum(x, 0.0) - x * t + p


def _sc_body(pred_hbm, target_hbm, out_hbm, buf, stage, sem0, sem1):
    wid = lax.axis_index("s") * NC + lax.axis_index("c")
    u0 = wid * UPW
    smp = u0 // TR
    row_base = (u0 - smp * TR) * 8
    sems = (sem0, sem1)
    zero = jnp.zeros((L,), jnp.float32)

    def issue(k):
        slot = k % 2
        r0 = row_base + k * 8
        return [
            pltpu.async_copy(
                pred_hbm.at[smp, pl.ds(1, 2), pl.ds(r0, 8), :],
                buf.at[slot, pl.ds(0, 2)], sems[slot]),
            pltpu.async_copy(
                target_hbm.at[smp, :, pl.ds(r0, 8), :],
                buf.at[slot, pl.ds(2, 3)], sems[slot]),
        ]

    inflight = {0: issue(0), 1: issue(1)}
    accs = (zero, zero, zero, zero)

    for k in range(UPW):
        slot = k % 2
        for h in inflight.pop(k):
            h.wait()
        if k + 2 < UPW:
            inflight[k + 2] = issue(k + 2)

        def row_step(r, carry, slot=slot):
            def step(j, carry, slot=slot, r=r):
                acc_t2, acc_c, acc_w, acc_wc = carry
                o = j * L
                p1 = buf[slot, 0, r, pl.ds(o, L)]
                p2 = buf[slot, 1, r, pl.ds(o, L)]
                t0 = buf[slot, 2, r, pl.ds(o, L)]
                t1 = buf[slot, 3, r, pl.ds(o, L)]
                t2 = buf[slot, 4, r, pl.ds(o, L)]
                tc = t2 - t2 * t0
                roi = 1.0 - tc * (1.0 - t1)
                wm = jnp.where(roi != 0.0, 1.0, 0.0)
                acc_t2 = acc_t2 + t2
                acc_c = acc_c + _bce(p2, tc)
                acc_w = acc_w + _bce(p1, t1) * wm
                acc_wc = acc_wc + wm
                return acc_t2, acc_c, acc_w, acc_wc

            return lax.fori_loop(0, NROWVEC, step, carry)

        accs = lax.fori_loop(0, 8, row_step, accs)

    for q in range(NQ):
        stage[q] = accs[q]
    pltpu.sync_copy(stage, out_hbm.at[wid])


def _tissue_body(p_ref, t_ref, o_ref):
    x = p_ref[...]
    t = t_ref[...]
    b = jnp.maximum(x, 0.0) - x * t + jnp.log1p(jnp.exp(-jnp.abs(x)))
    o_ref[0, 0] = jnp.sum(b)


def _combine_body(part_ref, tis_ref, o_ref):
    x = part_ref[...]
    t = jnp.sum(jnp.sum(x, axis=1), axis=-1)
    li = lax.broadcasted_iota(jnp.int32, (B, NQ), 1)
    sel = t[:, :1] != 0.0
    zero = jnp.zeros_like(t)
    cnum = jnp.sum(jnp.where((li == 1) & sel, t, zero))
    cden = jnp.sum(jnp.where((li == 1) & sel, jnp.float32(HW), zero))
    wsum = jnp.sum(jnp.where(li == 2, t, zero))
    wden = jnp.sum(jnp.where(li == 3, t, zero))
    centroid = jnp.where(cden > 0.0, cnum / jnp.maximum(cden, 1.0), 0.0)
    tissue = tis_ref[0, 0] / jnp.float32(B * HW)
    whole = jnp.where(wden > 0.0, wsum / jnp.maximum(wden, 1.0), 0.0)
    o_ref[0, 0] = whole + centroid + tissue


@jax.jit
def kernel(pred, target):
    mesh = plsc.VectorSubcoreMesh(core_axis_name="c", subcore_axis_name="s",
                                  num_cores=NC, num_subcores=NS)
    partials = pl.kernel(
        _sc_body,
        out_type=jax.ShapeDtypeStruct((NW, NQ, L), jnp.float32),
        mesh=mesh,
        scratch_types=[
            pltpu.VMEM((2, 5, 8, W), jnp.float32),
            pltpu.VMEM((NQ, L), jnp.float32),
            pltpu.SemaphoreType.DMA,
            pltpu.SemaphoreType.DMA,
        ],
        compiler_params=pltpu.CompilerParams(use_tc_tiling_on_sc=True),
    )(pred, target)
    tissue = pl.pallas_call(
        _tissue_body,
        grid=(1,),
        in_specs=[
            pl.BlockSpec((B, 1, H, W), lambda i: (0, 0, 0, 0)),
            pl.BlockSpec((B, 1, H, W), lambda i: (0, 0, 0, 0)),
        ],
        out_specs=pl.BlockSpec(memory_space=pltpu.SMEM),
        out_shape=jax.ShapeDtypeStruct((1, 1), jnp.float32),
    )(pred, target)
    out = pl.pallas_call(
        _combine_body,
        in_specs=[
            pl.BlockSpec((B, NW // B, NQ, L), lambda: (0, 0, 0, 0)),
            pl.BlockSpec(memory_space=pltpu.SMEM),
        ],
        out_specs=pl.BlockSpec(memory_space=pltpu.SMEM),
        out_shape=jax.ShapeDtypeStruct((1, 1), jnp.float32),
    )(partials.reshape(B, NW // B, NQ, L), tissue)
    return out[0, 0]


# speedup vs baseline: 1.1574x; 1.0393x over previous
import jax
import jax.numpy as jnp
from jax import lax
from jax.experimental import pallas as pl
from jax.experimental.pallas import tpu as pltpu
from jax.experimental.pallas import tpu_sc as plsc

L = 16
NC = 2
NS = 16
NW = NC * NS
B = 8
H = 224
W = 224
HW = H * W
TR = H // 8
UPW = B * TR // NW
NROWVEC = W // L
NQ = 4

_LOG1P_COEF = (
    -0.006006605050865348, 0.03426459995555095, -0.09229041738055285,
    0.16499812983410006, -0.23943337074600235, 0.33144665224343317,
    -0.49982549864347925, 0.9999936302585147, 3.910905554960209e-08,
)


def _bce(x, t):
    e = jnp.exp(-jnp.abs(x))
    p = jnp.float32(_LOG1P_COEF[0])
    for c in _LOG1P_COEF[1:]:
        p = p * e + jnp.float32(c)
    return jnp.maximum(x, 0.0) - x * t + p


def _sc_body(pred_hbm, target_hbm, out_hbm, buf, stage, sem):
    wid = lax.axis_index("s") * NC + lax.axis_index("c")
    u0 = wid * UPW
    smp = u0 // TR
    row_base = (u0 - smp * TR) * 8
    zero = jnp.zeros((L,), jnp.float32)

    def issue(k):
        slot = lax.rem(k, 2)
        r0 = row_base + k * 8
        pltpu.async_copy(
            pred_hbm.at[smp, pl.ds(1, 2), pl.ds(r0, 8), :],
            buf.at[slot, pl.ds(0, 2)], sem.at[slot])
        pltpu.async_copy(
            target_hbm.at[smp, :, pl.ds(r0, 8), :],
            buf.at[slot, pl.ds(2, 3)], sem.at[slot])

    issue(0)
    issue(1)

    def unit(k, carry):
        slot = lax.rem(k, 2)
        pltpu.make_async_copy(
            pred_hbm.at[smp, pl.ds(1, 2), pl.ds(0, 8), :],
            buf.at[slot, pl.ds(0, 2)], sem.at[slot]).wait()
        pltpu.make_async_copy(
            target_hbm.at[smp, :, pl.ds(0, 8), :],
            buf.at[slot, pl.ds(2, 3)], sem.at[slot]).wait()

        @pl.when(k + 2 < UPW)
        def _():
            issue(k + 2)

        def row_step(r, carry):
            def step(j, carry, r=r):
                acc_t2, acc_c, acc_w, acc_wc = carry
                o = j * L
                p1 = buf[slot, 0, r, pl.ds(o, L)]
                p2 = buf[slot, 1, r, pl.ds(o, L)]
                t0 = buf[slot, 2, r, pl.ds(o, L)]
                t1 = buf[slot, 3, r, pl.ds(o, L)]
                t2 = buf[slot, 4, r, pl.ds(o, L)]
                tc = t2 - t2 * t0
                roi = 1.0 - tc * (1.0 - t1)
                wm = jnp.where(roi != 0.0, 1.0, 0.0)
                acc_t2 = acc_t2 + t2
                acc_c = acc_c + _bce(p2, tc)
                acc_w = acc_w + _bce(p1, t1) * wm
                acc_wc = acc_wc + wm
                return acc_t2, acc_c, acc_w, acc_wc

            return lax.fori_loop(0, NROWVEC, step, carry)

        return lax.fori_loop(0, 8, row_step, carry)

    accs = lax.fori_loop(0, UPW, unit, (zero, zero, zero, zero))

    for q in range(NQ):
        stage[q] = accs[q]
    pltpu.sync_copy(stage, out_hbm.at[wid])


def _tissue_body(p_ref, t_ref, o_ref):
    x = p_ref[...]
    t = t_ref[...]
    b = jnp.maximum(x, 0.0) - x * t + jnp.log1p(jnp.exp(-jnp.abs(x)))
    o_ref[0, 0] = jnp.sum(b)


def _combine_body(part_ref, tis_ref, o_ref):
    x = part_ref[...]
    t = jnp.sum(jnp.sum(x, axis=1), axis=-1)
    li = lax.broadcasted_iota(jnp.int32, (B, NQ), 1)
    sel = t[:, :1] != 0.0
    zero = jnp.zeros_like(t)
    cnum = jnp.sum(jnp.where((li == 1) & sel, t, zero))
    cden = jnp.sum(jnp.where((li == 1) & sel, jnp.float32(HW), zero))
    wsum = jnp.sum(jnp.where(li == 2, t, zero))
    wden = jnp.sum(jnp.where(li == 3, t, zero))
    centroid = jnp.where(cden > 0.0, cnum / jnp.maximum(cden, 1.0), 0.0)
    tissue = tis_ref[0, 0] / jnp.float32(B * HW)
    whole = jnp.where(wden > 0.0, wsum / jnp.maximum(wden, 1.0), 0.0)
    o_ref[0, 0] = whole + centroid + tissue


@jax.jit
def kernel(pred, target):
    mesh = plsc.VectorSubcoreMesh(core_axis_name="c", subcore_axis_name="s",
                                  num_cores=NC, num_subcores=NS)
    partials = pl.kernel(
        _sc_body,
        out_type=jax.ShapeDtypeStruct((NW, NQ, L), jnp.float32),
        mesh=mesh,
        scratch_types=[
            pltpu.VMEM((2, 5, 8, W), jnp.float32),
            pltpu.VMEM((NQ, L), jnp.float32),
            pltpu.SemaphoreType.DMA((2,)),
        ],
        compiler_params=pltpu.CompilerParams(use_tc_tiling_on_sc=True),
    )(pred, target)
    tissue = pl.pallas_call(
        _tissue_body,
        grid=(1,),
        in_specs=[
            pl.BlockSpec((B, 1, H, W), lambda i: (0, 0, 0, 0)),
            pl.BlockSpec((B, 1, H, W), lambda i: (0, 0, 0, 0)),
        ],
        out_specs=pl.BlockSpec(memory_space=pltpu.SMEM),
        out_shape=jax.ShapeDtypeStruct((1, 1), jnp.float32),
    )(pred, target)
    out = pl.pallas_call(
        _combine_body,
        in_specs=[
            pl.BlockSpec((B, NW // B, NQ, L), lambda: (0, 0, 0, 0)),
            pl.BlockSpec(memory_space=pltpu.SMEM),
        ],
        out_specs=pl.BlockSpec(memory_space=pltpu.SMEM),
        out_shape=jax.ShapeDtypeStruct((1, 1), jnp.float32),
    )(partials.reshape(B, NW // B, NQ, L), tissue)
    return out[0, 0]


# speedup vs baseline: 1.3572x; 1.1727x over previous
import jax
import jax.numpy as jnp
from jax import lax
from jax.experimental import pallas as pl
from jax.experimental.pallas import tpu as pltpu
from jax.experimental.pallas import tpu_sc as plsc

L = 16
NC = 2
NS = 16
NW = NC * NS
B = 8
H = 224
W = 224
HW = H * W
TR = H // 8
UPW = B * TR // NW
NROWVEC = W // L
NQ = 2

_LOG1P_COEF = (
    -0.006006605050865348, 0.03426459995555095, -0.09229041738055285,
    0.16499812983410006, -0.23943337074600235, 0.33144665224343317,
    -0.49982549864347925, 0.9999936302585147, 3.910905554960209e-08,
)


def _bce(x, t):
    e = jnp.exp(-jnp.abs(x))
    p = jnp.float32(_LOG1P_COEF[0])
    for c in _LOG1P_COEF[1:]:
        p = p * e + jnp.float32(c)
    return jnp.maximum(x, 0.0) - x * t + p


def _sc_body(pred_hbm, target_hbm, out_hbm, buf, stage, sem):
    wid = lax.axis_index("s") * NC + lax.axis_index("c")
    u0 = wid * UPW
    smp = u0 // TR
    row_base = (u0 - smp * TR) * 8
    zero = jnp.zeros((L,), jnp.float32)

    def issue(k):
        slot = lax.rem(k, 2)
        r0 = row_base + k * 8
        pltpu.async_copy(
            pred_hbm.at[smp, 2, pl.ds(r0, 8), :],
            buf.at[slot, 0], sem.at[slot])
        pltpu.async_copy(
            target_hbm.at[smp, 0, pl.ds(r0, 8), :],
            buf.at[slot, 1], sem.at[slot])
        pltpu.async_copy(
            target_hbm.at[smp, 2, pl.ds(r0, 8), :],
            buf.at[slot, 2], sem.at[slot])

    issue(0)
    issue(1)

    def unit(k, carry):
        slot = lax.rem(k, 2)
        for d in range(3):
            pltpu.make_async_copy(
                pred_hbm.at[smp, 2, pl.ds(0, 8), :],
                buf.at[slot, d], sem.at[slot]).wait()

        @pl.when(k + 2 < UPW)
        def _():
            issue(k + 2)

        def row_step(r, carry):
            def step(j, carry, r=r):
                acc_t2, acc_c = carry
                o = j * L
                p2 = buf[slot, 0, r, pl.ds(o, L)]
                t0 = buf[slot, 1, r, pl.ds(o, L)]
                t2 = buf[slot, 2, r, pl.ds(o, L)]
                tc = t2 - t2 * t0
                acc_t2 = acc_t2 + t2
                acc_c = acc_c + _bce(p2, tc)
                return acc_t2, acc_c

            return lax.fori_loop(0, NROWVEC, step, carry)

        return lax.fori_loop(0, 8, row_step, carry)

    accs = lax.fori_loop(0, UPW, unit, (zero, zero))

    for q in range(NQ):
        stage[q] = accs[q]
    pltpu.sync_copy(stage, out_hbm.at[wid])


def _dense_body(p_ref, t_ref, o_ref):
    p0 = p_ref[:, 0]
    p1 = p_ref[:, 1]
    t0 = t_ref[:, 0]
    t1 = t_ref[:, 1]
    t2 = t_ref[:, 2]

    def bce(x, t):
        return jnp.maximum(x, 0.0) - x * t + jnp.log1p(jnp.exp(-jnp.abs(x)))

    roi = 1.0 - (t2 - t2 * t0) * (1.0 - t1)
    wm = jnp.where(roi != 0.0, 1.0, 0.0)
    o_ref[0, 0] = jnp.sum(bce(p0, t0))
    o_ref[0, 1] = jnp.sum(bce(p1, t1) * wm)
    o_ref[0, 2] = jnp.sum(wm)


def _combine_body(part_ref, dense_ref, o_ref):
    x = part_ref[...]
    t = jnp.sum(jnp.sum(x, axis=1), axis=-1)
    li = lax.broadcasted_iota(jnp.int32, (B, NQ), 1)
    sel = t[:, :1] != 0.0
    zero = jnp.zeros_like(t)
    cnum = jnp.sum(jnp.where((li == 1) & sel, t, zero))
    cden = jnp.sum(jnp.where((li == 1) & sel, jnp.float32(HW), zero))
    centroid = jnp.where(cden > 0.0, cnum / jnp.maximum(cden, 1.0), 0.0)
    tissue = dense_ref[0, 0] / jnp.float32(B * HW)
    wsum = dense_ref[0, 1]
    wden = dense_ref[0, 2]
    whole = jnp.where(wden > 0.0, wsum / jnp.maximum(wden, 1.0), 0.0)
    o_ref[0, 0] = whole + centroid + tissue


@jax.jit
def kernel(pred, target):
    mesh = plsc.VectorSubcoreMesh(core_axis_name="c", subcore_axis_name="s",
                                  num_cores=NC, num_subcores=NS)
    partials = pl.kernel(
        _sc_body,
        out_type=jax.ShapeDtypeStruct((NW, NQ, L), jnp.float32),
        mesh=mesh,
        scratch_types=[
            pltpu.VMEM((2, 3, 8, W), jnp.float32),
            pltpu.VMEM((NQ, L), jnp.float32),
            pltpu.SemaphoreType.DMA((2,)),
        ],
        compiler_params=pltpu.CompilerParams(use_tc_tiling_on_sc=True),
    )(pred, target)
    dense = pl.pallas_call(
        _dense_body,
        grid=(1,),
        in_specs=[
            pl.BlockSpec((B, 2, H, W), lambda i: (0, 0, 0, 0)),
            pl.BlockSpec((B, 3, H, W), lambda i: (0, 0, 0, 0)),
        ],
        out_specs=pl.BlockSpec(memory_space=pltpu.SMEM),
        out_shape=jax.ShapeDtypeStruct((1, 3), jnp.float32),
    )(pred, target)
    out = pl.pallas_call(
        _combine_body,
        in_specs=[
            pl.BlockSpec((B, NW // B, NQ, L), lambda: (0, 0, 0, 0)),
            pl.BlockSpec(memory_space=pltpu.SMEM),
        ],
        out_specs=pl.BlockSpec(memory_space=pltpu.SMEM),
        out_shape=jax.ShapeDtypeStruct((1, 1), jnp.float32),
    )(partials.reshape(B, NW // B, NQ, L), dense)
    return out[0, 0]


# speedup vs baseline: 1.3777x; 1.0151x over previous
import jax
import jax.numpy as jnp
from jax import lax
from jax.experimental import pallas as pl
from jax.experimental.pallas import tpu as pltpu
from jax.experimental.pallas import tpu_sc as plsc

L = 16
NC = 2
NS = 16
NW = NC * NS
B = 8
H = 224
W = 224
HW = H * W
TR = H // 8
UPW = B * TR // NW
NROWVEC = W // L
NQ = 2

_LOG1P_COEF = (
    -0.006006605050865348, 0.03426459995555095, -0.09229041738055285,
    0.16499812983410006, -0.23943337074600235, 0.33144665224343317,
    -0.49982549864347925, 0.9999936302585147, 3.910905554960209e-08,
)


def _bce(x, t):
    e = jnp.exp(-jnp.abs(x))
    p = jnp.float32(_LOG1P_COEF[0])
    for c in _LOG1P_COEF[1:]:
        p = p * e + jnp.float32(c)
    return jnp.maximum(x, 0.0) - x * t + p


def _sc_body(pred_hbm, target_hbm, out_hbm, buf, stage, sem):
    wid = lax.axis_index("s") * NC + lax.axis_index("c")
    u0 = wid * UPW
    smp = u0 // TR
    row_base = (u0 - smp * TR) * 8
    zero = jnp.zeros((L,), jnp.float32)

    def issue(k):
        slot = lax.rem(k, 2)
        r0 = row_base + k * 8
        pltpu.async_copy(
            pred_hbm.at[smp, 2, pl.ds(r0, 8), :],
            buf.at[slot, 0], sem.at[slot])
        pltpu.async_copy(
            target_hbm.at[smp, 0, pl.ds(r0, 8), :],
            buf.at[slot, 1], sem.at[slot])
        pltpu.async_copy(
            target_hbm.at[smp, 2, pl.ds(r0, 8), :],
            buf.at[slot, 2], sem.at[slot])

    issue(0)
    issue(1)

    def unit(k, carry):
        slot = lax.rem(k, 2)
        for d in range(3):
            pltpu.make_async_copy(
                pred_hbm.at[smp, 2, pl.ds(0, 8), :],
                buf.at[slot, d], sem.at[slot]).wait()

        @pl.when(k + 2 < UPW)
        def _():
            issue(k + 2)

        def row_step(r, carry):
            def step(j, carry, r=r):
                acc_t2, acc_c = carry
                o = j * L
                p2 = buf[slot, 0, r, pl.ds(o, L)]
                t0 = buf[slot, 1, r, pl.ds(o, L)]
                t2 = buf[slot, 2, r, pl.ds(o, L)]
                tc = t2 - t2 * t0
                acc_t2 = acc_t2 + t2
                acc_c = acc_c + _bce(p2, tc)
                return acc_t2, acc_c

            return lax.fori_loop(0, NROWVEC, step, carry)

        return lax.fori_loop(0, 8, row_step, carry)

    accs = lax.fori_loop(0, UPW, unit, (zero, zero))

    for q in range(NQ):
        stage[q] = accs[q]
    pltpu.sync_copy(stage, out_hbm.at[wid])


def _dense_body(p_ref, t_ref, o_ref):
    i = pl.program_id(0)
    p0 = p_ref[0, 0]
    p1 = p_ref[0, 1]
    t0 = t_ref[0, 0]
    t1 = t_ref[0, 1]
    t2 = t_ref[0, 2]

    def bce(x, t):
        return jnp.maximum(x, 0.0) - x * t + jnp.log1p(jnp.exp(-jnp.abs(x)))

    roi = 1.0 - (t2 - t2 * t0) * (1.0 - t1)
    wm = jnp.where(roi != 0.0, 1.0, 0.0)

    @pl.when(i == 0)
    def _():
        o_ref[0, 0] = 0.0
        o_ref[0, 1] = 0.0
        o_ref[0, 2] = 0.0

    o_ref[0, 0] += jnp.sum(bce(p0, t0))
    o_ref[0, 1] += jnp.sum(bce(p1, t1) * wm)
    o_ref[0, 2] += jnp.sum(wm)


def _combine_body(part_ref, dense_ref, o_ref):
    x = part_ref[...]
    t = jnp.sum(jnp.sum(x, axis=1), axis=-1)
    li = lax.broadcasted_iota(jnp.int32, (B, NQ), 1)
    sel = t[:, :1] != 0.0
    zero = jnp.zeros_like(t)
    cnum = jnp.sum(jnp.where((li == 1) & sel, t, zero))
    cden = jnp.sum(jnp.where((li == 1) & sel, jnp.float32(HW), zero))
    centroid = jnp.where(cden > 0.0, cnum / jnp.maximum(cden, 1.0), 0.0)
    tissue = dense_ref[0, 0] / jnp.float32(B * HW)
    wsum = dense_ref[0, 1]
    wden = dense_ref[0, 2]
    whole = jnp.where(wden > 0.0, wsum / jnp.maximum(wden, 1.0), 0.0)
    o_ref[0, 0] = whole + centroid + tissue


@jax.jit
def kernel(pred, target):
    mesh = plsc.VectorSubcoreMesh(core_axis_name="c", subcore_axis_name="s",
                                  num_cores=NC, num_subcores=NS)
    partials = pl.kernel(
        _sc_body,
        out_type=jax.ShapeDtypeStruct((NW, NQ, L), jnp.float32),
        mesh=mesh,
        scratch_types=[
            pltpu.VMEM((2, 3, 8, W), jnp.float32),
            pltpu.VMEM((NQ, L), jnp.float32),
            pltpu.SemaphoreType.DMA((2,)),
        ],
        compiler_params=pltpu.CompilerParams(use_tc_tiling_on_sc=True),
    )(pred, target)
    dense = pl.pallas_call(
        _dense_body,
        grid=(B,),
        in_specs=[
            pl.BlockSpec((1, 2, H, W), lambda i: (i, 0, 0, 0)),
            pl.BlockSpec((1, 3, H, W), lambda i: (i, 0, 0, 0)),
        ],
        out_specs=pl.BlockSpec((1, 3), lambda i: (0, 0), memory_space=pltpu.SMEM),
        out_shape=jax.ShapeDtypeStruct((1, 3), jnp.float32),
    )(pred, target)
    out = pl.pallas_call(
        _combine_body,
        in_specs=[
            pl.BlockSpec((B, NW // B, NQ, L), lambda: (0, 0, 0, 0)),
            pl.BlockSpec(memory_space=pltpu.SMEM),
        ],
        out_specs=pl.BlockSpec(memory_space=pltpu.SMEM),
        out_shape=jax.ShapeDtypeStruct((1, 1), jnp.float32),
    )(partials.reshape(B, NW // B, NQ, L), dense)
    return out[0, 0]


# speedup vs baseline: 1.4636x; 1.0623x over previous
import jax
import jax.numpy as jnp
from jax import lax
from jax.experimental import pallas as pl
from jax.experimental.pallas import tpu as pltpu
from jax.experimental.pallas import tpu_sc as plsc

L = 16
NC = 2
NS = 16
NW = NC * NS
B = 8
H = 224
W = 224
HW = H * W
TR = H // 8
UPW = 5
SC_ROWS = UPW * (NW // B) * 8
NROWVEC = W // L
NQ = 2

_LOG1P_COEF = (
    -0.006006605050865348, 0.03426459995555095, -0.09229041738055285,
    0.16499812983410006, -0.23943337074600235, 0.33144665224343317,
    -0.49982549864347925, 0.9999936302585147, 3.910905554960209e-08,
)


def _bce(x, t):
    e = jnp.exp(-jnp.abs(x))
    p = jnp.float32(_LOG1P_COEF[0])
    for c in _LOG1P_COEF[1:]:
        p = p * e + jnp.float32(c)
    return jnp.maximum(x, 0.0) - x * t + p


def _sc_body(pred_hbm, target_hbm, out_hbm, buf, stage, sem):
    wid = lax.axis_index("s") * NC + lax.axis_index("c")
    u0 = wid * UPW
    ups = UPW * (NW // B)
    smp = u0 // ups
    row_base = (u0 - smp * ups) * 8
    zero = jnp.zeros((L,), jnp.float32)

    def issue(k):
        slot = lax.rem(k, 2)
        r0 = row_base + k * 8
        pltpu.async_copy(
            pred_hbm.at[smp, 2, pl.ds(r0, 8), :],
            buf.at[slot, 0], sem.at[slot])
        pltpu.async_copy(
            target_hbm.at[smp, 0, pl.ds(r0, 8), :],
            buf.at[slot, 1], sem.at[slot])
        pltpu.async_copy(
            target_hbm.at[smp, 2, pl.ds(r0, 8), :],
            buf.at[slot, 2], sem.at[slot])

    issue(0)
    issue(1)

    def unit(k, carry):
        slot = lax.rem(k, 2)
        for d in range(3):
            pltpu.make_async_copy(
                pred_hbm.at[smp, 2, pl.ds(0, 8), :],
                buf.at[slot, d], sem.at[slot]).wait()

        @pl.when(k + 2 < UPW)
        def _():
            issue(k + 2)

        def row_step(r, carry):
            def step(j, carry, r=r):
                acc_t2, acc_c = carry
                o = j * L
                p2 = buf[slot, 0, r, pl.ds(o, L)]
                t0 = buf[slot, 1, r, pl.ds(o, L)]
                t2 = buf[slot, 2, r, pl.ds(o, L)]
                tc = t2 - t2 * t0
                acc_t2 = acc_t2 + t2
                acc_c = acc_c + _bce(p2, tc)
                return acc_t2, acc_c

            return lax.fori_loop(0, NROWVEC, step, carry)

        return lax.fori_loop(0, 8, row_step, carry)

    accs = lax.fori_loop(0, UPW, unit, (zero, zero))

    for q in range(NQ):
        stage[q] = accs[q]
    pltpu.sync_copy(stage, out_hbm.at[wid])


def _dense_body(p_ref, t_ref, o_ref, tail_ref):
    i = pl.program_id(0)
    p0 = p_ref[0, 0]
    p1 = p_ref[0, 1]
    t0 = t_ref[0, 0]
    t1 = t_ref[0, 1]
    t2 = t_ref[0, 2]

    def bce(x, t):
        return jnp.maximum(x, 0.0) - x * t + jnp.log1p(jnp.exp(-jnp.abs(x)))

    roi = 1.0 - (t2 - t2 * t0) * (1.0 - t1)
    wm = jnp.where(roi != 0.0, 1.0, 0.0)

    @pl.when(i == 0)
    def _():
        o_ref[0, 0] = 0.0
        o_ref[0, 1] = 0.0
        o_ref[0, 2] = 0.0

    o_ref[0, 0] += jnp.sum(bce(p0, t0))
    o_ref[0, 1] += jnp.sum(bce(p1, t1) * wm)
    o_ref[0, 2] += jnp.sum(wm)

    p2t = p_ref[0, 2, SC_ROWS:, :]
    t0t = t0[SC_ROWS:, :]
    t2t = t2[SC_ROWS:, :]
    tct = t2t - t2t * t0t
    tail_ref[...] = jnp.reshape(
        jnp.stack([jnp.sum(t2t), jnp.sum(bce(p2t, tct))]), (1, 1, 2))


def _combine_body(part_ref, dense_ref, tail_ref, o_ref):
    x = part_ref[...]
    t = jnp.sum(jnp.sum(x, axis=1), axis=-1) + tail_ref[...][:, 0, :]
    li = lax.broadcasted_iota(jnp.int32, (B, NQ), 1)
    sel = t[:, :1] != 0.0
    zero = jnp.zeros_like(t)
    cnum = jnp.sum(jnp.where((li == 1) & sel, t, zero))
    cden = jnp.sum(jnp.where((li == 1) & sel, jnp.float32(HW), zero))
    centroid = jnp.where(cden > 0.0, cnum / jnp.maximum(cden, 1.0), 0.0)
    tissue = dense_ref[0, 0] / jnp.float32(B * HW)
    wsum = dense_ref[0, 1]
    wden = dense_ref[0, 2]
    whole = jnp.where(wden > 0.0, wsum / jnp.maximum(wden, 1.0), 0.0)
    o_ref[0, 0] = whole + centroid + tissue


@jax.jit
def kernel(pred, target):
    mesh = plsc.VectorSubcoreMesh(core_axis_name="c", subcore_axis_name="s",
                                  num_cores=NC, num_subcores=NS)
    partials = pl.kernel(
        _sc_body,
        out_type=jax.ShapeDtypeStruct((NW, NQ, L), jnp.float32),
        mesh=mesh,
        scratch_types=[
            pltpu.VMEM((2, 3, 8, W), jnp.float32),
            pltpu.VMEM((NQ, L), jnp.float32),
            pltpu.SemaphoreType.DMA((2,)),
        ],
        compiler_params=pltpu.CompilerParams(use_tc_tiling_on_sc=True),
    )(pred, target)
    dense, tails = pl.pallas_call(
        _dense_body,
        grid=(B,),
        in_specs=[
            pl.BlockSpec((1, 3, H, W), lambda i: (i, 0, 0, 0)),
            pl.BlockSpec((1, 3, H, W), lambda i: (i, 0, 0, 0)),
        ],
        out_specs=[
            pl.BlockSpec((1, 3), lambda i: (0, 0), memory_space=pltpu.SMEM),
            pl.BlockSpec((1, 1, 2), lambda i: (i, 0, 0)),
        ],
        out_shape=[
            jax.ShapeDtypeStruct((1, 3), jnp.float32),
            jax.ShapeDtypeStruct((B, 1, 2), jnp.float32),
        ],
    )(pred, target)
    out = pl.pallas_call(
        _combine_body,
        in_specs=[
            pl.BlockSpec((B, NW // B, NQ, L), lambda: (0, 0, 0, 0)),
            pl.BlockSpec(memory_space=pltpu.SMEM),
            pl.BlockSpec((B, 1, 2), lambda: (0, 0, 0)),
        ],
        out_specs=pl.BlockSpec(memory_space=pltpu.SMEM),
        out_shape=jax.ShapeDtypeStruct((1, 1), jnp.float32),
    )(partials.reshape(B, NW // B, NQ, L), dense, tails)
    return out[0, 0]


# speedup vs baseline: 1.4656x; 1.0014x over previous
import jax
import jax.numpy as jnp
from jax import lax
from jax.experimental import pallas as pl
from jax.experimental.pallas import tpu as pltpu
from jax.experimental.pallas import tpu_sc as plsc

L = 16
NC = 2
NS = 16
NW = NC * NS
B = 8
H = 224
W = 224
HW = H * W
TR = H // 8
UPW = 5
SC_ROWS = UPW * (NW // B) * 8
NROWVEC = W // L
NQ = 2

_LOG1P_COEF = (
    -0.006006605050865348, 0.03426459995555095, -0.09229041738055285,
    0.16499812983410006, -0.23943337074600235, 0.33144665224343317,
    -0.49982549864347925, 0.9999936302585147, 3.910905554960209e-08,
)


def _bce(x, t):
    e = jnp.exp(-jnp.abs(x))
    p = jnp.float32(_LOG1P_COEF[0])
    for c in _LOG1P_COEF[1:]:
        p = p * e + jnp.float32(c)
    return jnp.maximum(x, 0.0) - x * t + p


def _sc_body(pred_hbm, target_hbm, out_hbm, buf, stage, sem):
    wid = lax.axis_index("s") * NC + lax.axis_index("c")
    u0 = wid * UPW
    ups = UPW * (NW // B)
    smp = u0 // ups
    row_base = (u0 - smp * ups) * 8
    zero = jnp.zeros((L,), jnp.float32)

    def issue(k):
        slot = lax.rem(k, 2)
        r0 = row_base + k * 8
        pltpu.async_copy(
            pred_hbm.at[smp, 2, pl.ds(r0, 8), :],
            buf.at[slot, 0], sem.at[slot])
        pltpu.async_copy(
            target_hbm.at[smp, 0, pl.ds(r0, 8), :],
            buf.at[slot, 1], sem.at[slot])
        pltpu.async_copy(
            target_hbm.at[smp, 2, pl.ds(r0, 8), :],
            buf.at[slot, 2], sem.at[slot])

    issue(0)
    issue(1)

    def unit(k, carry):
        slot = lax.rem(k, 2)
        for d in range(3):
            pltpu.make_async_copy(
                pred_hbm.at[smp, 2, pl.ds(0, 8), :],
                buf.at[slot, d], sem.at[slot]).wait()

        @pl.when(k + 2 < UPW)
        def _():
            issue(k + 2)

        def row_step(r, carry):
            def step(j, carry, r=r):
                acc_t2, acc_c = carry
                o = j * L
                p2 = buf[slot, 0, r, pl.ds(o, L)]
                t0 = buf[slot, 1, r, pl.ds(o, L)]
                t2 = buf[slot, 2, r, pl.ds(o, L)]
                tc = t2 - t2 * t0
                acc_t2 = acc_t2 + t2
                acc_c = acc_c + _bce(p2, tc)
                return acc_t2, acc_c

            return lax.fori_loop(0, NROWVEC, step, carry)

        return lax.fori_loop(0, 8, row_step, carry)

    accs = lax.fori_loop(0, UPW, unit, (zero, zero))

    for q in range(NQ):
        stage[q] = accs[q]
    pltpu.sync_copy(stage, out_hbm.at[wid])


def _dense_body(p_ref, t_ref, o_ref, tail_ref):
    i = pl.program_id(0)
    p0 = p_ref[0, 0]
    p1 = p_ref[0, 1]
    t0 = t_ref[0, 0]
    t1 = t_ref[0, 1]
    t2 = t_ref[0, 2]

    def bce(x, t):
        return jnp.maximum(x, 0.0) - x * t + jnp.log1p(jnp.exp(-jnp.abs(x)))

    roi = 1.0 - (t2 - t2 * t0) * (1.0 - t1)
    wm = jnp.where(roi != 0.0, 1.0, 0.0)

    @pl.when(i == 0)
    def _():
        o_ref[0, 0] = 0.0
        o_ref[0, 1] = 0.0
        o_ref[0, 2] = 0.0

    o_ref[0, 0] += jnp.sum(bce(p0, t0))
    o_ref[0, 1] += jnp.sum(bce(p1, t1) * wm)
    o_ref[0, 2] += jnp.sum(wm)

    p2t = p_ref[0, 2, SC_ROWS:, :]
    t0t = t0[SC_ROWS:, :]
    t2t = t2[SC_ROWS:, :]
    tct = t2t - t2t * t0t
    tail_ref[...] = jnp.reshape(
        jnp.stack([jnp.sum(t2t), jnp.sum(bce(p2t, tct))]), (1, 1, 2))


def _combine_body(part_ref, dense_ref, tail_ref, o_ref):
    x = part_ref[...]
    t = jnp.sum(jnp.sum(x, axis=1), axis=-1) + tail_ref[...][:, 0, :]
    li = lax.broadcasted_iota(jnp.int32, (B, NQ), 1)
    sel = t[:, :1] != 0.0
    zero = jnp.zeros_like(t)
    cnum = jnp.sum(jnp.where((li == 1) & sel, t, zero))
    cden = jnp.sum(jnp.where((li == 1) & sel, jnp.float32(HW), zero))
    centroid = jnp.where(cden > 0.0, cnum / jnp.maximum(cden, 1.0), 0.0)
    tissue = dense_ref[0, 0] / jnp.float32(B * HW)
    wsum = dense_ref[0, 1]
    wden = dense_ref[0, 2]
    whole = jnp.where(wden > 0.0, wsum / jnp.maximum(wden, 1.0), 0.0)
    o_ref[0, 0] = whole + centroid + tissue


@jax.jit
def kernel(pred, target):
    mesh = plsc.VectorSubcoreMesh(core_axis_name="c", subcore_axis_name="s",
                                  num_cores=NC, num_subcores=NS)
    partials = pl.kernel(
        _sc_body,
        out_type=jax.ShapeDtypeStruct((NW, NQ, L), jnp.float32),
        mesh=mesh,
        scratch_types=[
            pltpu.VMEM((2, 3, 8, W), jnp.float32),
            pltpu.VMEM((NQ, L), jnp.float32),
            pltpu.SemaphoreType.DMA((2,)),
        ],
        compiler_params=pltpu.CompilerParams(use_tc_tiling_on_sc=True, skip_device_barrier=True),
    )(pred, target)
    dense, tails = pl.pallas_call(
        _dense_body,
        grid=(B,),
        in_specs=[
            pl.BlockSpec((1, 3, H, W), lambda i: (i, 0, 0, 0)),
            pl.BlockSpec((1, 3, H, W), lambda i: (i, 0, 0, 0)),
        ],
        out_specs=[
            pl.BlockSpec((1, 3), lambda i: (0, 0), memory_space=pltpu.SMEM),
            pl.BlockSpec((1, 1, 2), lambda i: (i, 0, 0)),
        ],
        out_shape=[
            jax.ShapeDtypeStruct((1, 3), jnp.float32),
            jax.ShapeDtypeStruct((B, 1, 2), jnp.float32),
        ],
    )(pred, target)
    out = pl.pallas_call(
        _combine_body,
        in_specs=[
            pl.BlockSpec((B, NW // B, NQ, L), lambda: (0, 0, 0, 0)),
            pl.BlockSpec(memory_space=pltpu.SMEM),
            pl.BlockSpec((B, 1, 2), lambda: (0, 0, 0)),
        ],
        out_specs=pl.BlockSpec(memory_space=pltpu.SMEM),
        out_shape=jax.ShapeDtypeStruct((1, 1), jnp.float32),
    )(partials.reshape(B, NW // B, NQ, L), dense, tails)
    return out[0, 0]


# speedup vs baseline: 1.4705x; 1.0033x over previous
import jax
import jax.numpy as jnp
from jax import lax
from jax.experimental import pallas as pl
from jax.experimental.pallas import tpu as pltpu
from jax.experimental.pallas import tpu_sc as plsc

L = 16
NC = 2
NS = 16
NW = NC * NS
B = 8
H = 224
W = 224
HW = H * W
TR = H // 8
UPW = 5
SC_ROWS = UPW * (NW // B) * 8
NROWVEC = W // L
NQ = 2

_LOG1P_COEF = (
    -0.006006605050865348, 0.03426459995555095, -0.09229041738055285,
    0.16499812983410006, -0.23943337074600235, 0.33144665224343317,
    -0.49982549864347925, 0.9999936302585147, 3.910905554960209e-08,
)


def _bce(x, t):
    e = jnp.exp(-jnp.abs(x))
    p = jnp.float32(_LOG1P_COEF[0])
    for c in _LOG1P_COEF[1:]:
        p = p * e + jnp.float32(c)
    return jnp.maximum(x, 0.0) - x * t + p


def _sc_body(pred_hbm, target_hbm, out_hbm, buf, stage, sem):
    wid = lax.axis_index("s") * NC + lax.axis_index("c")
    u0 = wid * UPW
    ups = UPW * (NW // B)
    smp = u0 // ups
    row_base = (u0 - smp * ups) * 8
    zero = jnp.zeros((L,), jnp.float32)

    def issue(k):
        slot = lax.rem(k, 2)
        r0 = row_base + k * 8
        pltpu.async_copy(
            pred_hbm.at[smp, 2, pl.ds(r0, 8), :],
            buf.at[slot, 0], sem.at[slot])
        pltpu.async_copy(
            target_hbm.at[smp, 0, pl.ds(r0, 8), :],
            buf.at[slot, 1], sem.at[slot])
        pltpu.async_copy(
            target_hbm.at[smp, 2, pl.ds(r0, 8), :],
            buf.at[slot, 2], sem.at[slot])

    issue(0)
    issue(1)

    def unit(k, carry):
        slot = lax.rem(k, 2)
        for d in range(3):
            pltpu.make_async_copy(
                pred_hbm.at[smp, 2, pl.ds(0, 8), :],
                buf.at[slot, d], sem.at[slot]).wait()

        @pl.when(k + 2 < UPW)
        def _():
            issue(k + 2)

        def row_step(r, carry):
            def step(j, carry, r=r):
                acc_t2, acc_c = carry
                for h in range(2):
                    o = (j * 2 + h) * L
                    p2 = buf[slot, 0, r, pl.ds(o, L)]
                    t0 = buf[slot, 1, r, pl.ds(o, L)]
                    t2 = buf[slot, 2, r, pl.ds(o, L)]
                    tc = t2 - t2 * t0
                    acc_t2 = acc_t2 + t2
                    acc_c = acc_c + _bce(p2, tc)
                return acc_t2, acc_c

            return lax.fori_loop(0, NROWVEC // 2, step, carry)

        return lax.fori_loop(0, 8, row_step, carry)

    accs = lax.fori_loop(0, UPW, unit, (zero, zero))

    for q in range(NQ):
        stage[q] = accs[q]
    pltpu.sync_copy(stage, out_hbm.at[wid])


def _dense_body(p_ref, t_ref, o_ref, tail_ref):
    i = pl.program_id(0)
    p0 = p_ref[0, 0]
    p1 = p_ref[0, 1]
    t0 = t_ref[0, 0]
    t1 = t_ref[0, 1]
    t2 = t_ref[0, 2]

    def bce(x, t):
        return jnp.maximum(x, 0.0) - x * t + jnp.log1p(jnp.exp(-jnp.abs(x)))

    roi = 1.0 - (t2 - t2 * t0) * (1.0 - t1)
    wm = jnp.where(roi != 0.0, 1.0, 0.0)

    @pl.when(i == 0)
    def _():
        o_ref[0, 0] = 0.0
        o_ref[0, 1] = 0.0
        o_ref[0, 2] = 0.0

    o_ref[0, 0] += jnp.sum(bce(p0, t0))
    o_ref[0, 1] += jnp.sum(bce(p1, t1) * wm)
    o_ref[0, 2] += jnp.sum(wm)

    p2t = p_ref[0, 2, SC_ROWS:, :]
    t0t = t0[SC_ROWS:, :]
    t2t = t2[SC_ROWS:, :]
    tct = t2t - t2t * t0t
    tail_ref[...] = jnp.reshape(
        jnp.stack([jnp.sum(t2t), jnp.sum(bce(p2t, tct))]), (1, 1, 2))


def _combine_body(part_ref, dense_ref, tail_ref, o_ref):
    x = part_ref[...]
    t = jnp.sum(jnp.sum(x, axis=1), axis=-1) + tail_ref[...][:, 0, :]
    li = lax.broadcasted_iota(jnp.int32, (B, NQ), 1)
    sel = t[:, :1] != 0.0
    zero = jnp.zeros_like(t)
    cnum = jnp.sum(jnp.where((li == 1) & sel, t, zero))
    cden = jnp.sum(jnp.where((li == 1) & sel, jnp.float32(HW), zero))
    centroid = jnp.where(cden > 0.0, cnum / jnp.maximum(cden, 1.0), 0.0)
    tissue = dense_ref[0, 0] / jnp.float32(B * HW)
    wsum = dense_ref[0, 1]
    wden = dense_ref[0, 2]
    whole = jnp.where(wden > 0.0, wsum / jnp.maximum(wden, 1.0), 0.0)
    o_ref[0, 0] = whole + centroid + tissue


@jax.jit
def kernel(pred, target):
    mesh = plsc.VectorSubcoreMesh(core_axis_name="c", subcore_axis_name="s",
                                  num_cores=NC, num_subcores=NS)
    partials = pl.kernel(
        _sc_body,
        out_type=jax.ShapeDtypeStruct((NW, NQ, L), jnp.float32),
        mesh=mesh,
        scratch_types=[
            pltpu.VMEM((2, 3, 8, W), jnp.float32),
            pltpu.VMEM((NQ, L), jnp.float32),
            pltpu.SemaphoreType.DMA((2,)),
        ],
        compiler_params=pltpu.CompilerParams(use_tc_tiling_on_sc=True),
    )(pred, target)
    dense, tails = pl.pallas_call(
        _dense_body,
        grid=(B,),
        in_specs=[
            pl.BlockSpec((1, 3, H, W), lambda i: (i, 0, 0, 0)),
            pl.BlockSpec((1, 3, H, W), lambda i: (i, 0, 0, 0)),
        ],
        out_specs=[
            pl.BlockSpec((1, 3), lambda i: (0, 0), memory_space=pltpu.SMEM),
            pl.BlockSpec((1, 1, 2), lambda i: (i, 0, 0)),
        ],
        out_shape=[
            jax.ShapeDtypeStruct((1, 3), jnp.float32),
            jax.ShapeDtypeStruct((B, 1, 2), jnp.float32),
        ],
    )(pred, target)
    out = pl.pallas_call(
        _combine_body,
        in_specs=[
            pl.BlockSpec((B, NW // B, NQ, L), lambda: (0, 0, 0, 0)),
            pl.BlockSpec(memory_space=pltpu.SMEM),
            pl.BlockSpec((B, 1, 2), lambda: (0, 0, 0)),
        ],
        out_specs=pl.BlockSpec(memory_space=pltpu.SMEM),
        out_shape=jax.ShapeDtypeStruct((1, 1), jnp.float32),
    )(partials.reshape(B, NW // B, NQ, L), dense, tails)
    return out[0, 0]
